# baseline (speedup 1.0000x reference)
import jax
import jax.numpy as jnp
from jax import lax
from jax.experimental import pallas as pl
from jax.experimental.pallas import tpu as pltpu

_MOM = 0.1
_B, _L, _K, _C = 8, 2048, 8, 16
_N = _B * _L
_W = 2048
_WPB = _L // _W
_GRID = _B * _WPB
_IPS = 32 // _GRID
_IMAX = 2147483647
_F32 = jnp.float32


def _body(seq_ref, pri_ref, thr_ref, l1_ref, l2_ref, lp_ref, ln_ref,
          mt_ref, pv_ref,
          f1_ref, f2_ref, po_ref, pro_ref, tho_ref, acc_ref, keys_ref,
          st_ref):
    g = pl.program_id(0)

    @pl.when(g == 0)
    def _init():
        acc_ref[...] = jnp.zeros_like(acc_ref)
        x = pv_ref[...]
        bits = lax.bitcast_convert_type(x, jnp.int32)
        keys = jnp.where(bits < 0, bits ^ jnp.int32(0x7FFFFFFF), bits)
        bio = lax.broadcasted_iota(jnp.int32, (_B, _K, _L), 0)
        lio3 = lax.broadcasted_iota(jnp.int32, (_B, _K, _L), 2)
        seqv = jnp.zeros((_B, _K, _L), jnp.int32)
        for b in range(_B):
            seqv = seqv + jnp.where(bio == b, seq_ref[b], 0)
        keys_ref[...] = jnp.where(lio3 < seqv, keys, _IMAX)
        st_ref[0:1] = jnp.full((1, _K, 1), jnp.int32(-2147483647) - 1)
        st_ref[1:2] = jnp.full((1, _K, 1), _IMAX, jnp.int32)

    m = mt_ref[0]
    x1, x2 = l1_ref[0], l2_ref[0]
    xp, xn = lp_ref[0], ln_ref[0]

    one = jnp.ones((), _F32)
    zero = jnp.zeros((), _F32)

    mask2 = m >= 0
    seq_b = seq_ref[g // _WPB]
    lio2 = (g % _WPB) * _W + lax.broadcasted_iota(jnp.int32, (_K, _W), 1)
    idx2 = lio2 < seq_b
    mi2 = mask2 & idx2

    cio = lax.broadcasted_iota(jnp.int32, (_K, _C, _W), 1)
    oh = m[:, None, :] == cio
    ohi = oh & idx2[:, None, :]

    acc_ref[0:_K, :] += jnp.sum(jnp.where(oh, x1, zero), axis=1)
    acc_ref[_K:2 * _K, :] += jnp.sum(jnp.where(oh, x2, zero), axis=1)
    pres = jnp.sum(jnp.where(ohi, xp, zero), axis=1) \
        - xn[:, 0, :] * jnp.where(idx2 & ~mask2, one, zero)
    acc_ref[2 * _K:3 * _K, :] += pres
    acc_ref[3 * _K:4 * _K, :] += jnp.where(mask2, one, zero)
    acc_ref[4 * _K:5 * _K, :] += jnp.where(mi2, one, zero)

    cnt_total = jnp.int32(0)
    for b in range(_B):
        cnt_total = cnt_total + jnp.minimum(seq_ref[b], _L)
    ic = cnt_total.astype(_F32)
    kio = lax.broadcasted_iota(jnp.int32, (1, _K, 1), 1)
    priv2 = jnp.zeros((1, _K, 1), _F32)
    for k in range(_K):
        priv2 = priv2 + pri_ref[k] * jnp.where(kio == k, 1.0, 0.0)
    ind = (1.0 - priv2) * ic
    nm1 = cnt_total - 1
    rb = jnp.clip(jnp.floor(ind).astype(jnp.int32), 0, nm1)
    rbf = (rb + 1).astype(_F32)

    def _cnt(thr):
        sel = jnp.where(keys_ref[...] <= thr, one, zero)
        s = jnp.sum(sel, axis=2, keepdims=True)
        return jnp.sum(s, axis=0, keepdims=True)

    lo = st_ref[0:1]
    hi = st_ref[1:2]
    for _ in range(_IPS):
        mid = (lo >> 1) + (hi >> 1) + (lo & hi & 1)
        pred = _cnt(mid) >= rbf
        lo = jnp.where(pred, lo, mid + 1)
        hi = jnp.where(pred, mid, hi)
    st_ref[0:1] = lo
    st_ref[1:2] = hi

    @pl.when(g == _GRID - 1)
    def _fin():
        s1 = jnp.sum(acc_ref[0:_K, :])
        s2 = jnp.sum(acc_ref[_K:2 * _K, :])
        sp = jnp.sum(acc_ref[2 * _K:3 * _K, :])
        mc = jnp.sum(acc_ref[3 * _K:4 * _K, :])
        mcount = jnp.maximum(mc, 1.0)
        icount = jnp.maximum(ic * _K, 1.0)
        f1_ref[...] = jnp.full((1, 1), s1 / mcount, _F32)
        f2_ref[...] = jnp.full((1, 1), s2 / mcount, _F32)
        po_ref[...] = jnp.full((1, 1), sp / icount, _F32)

        kcnt = jnp.sum(acc_ref[4 * _K:5 * _K, :], axis=1, keepdims=True)
        sio8 = lax.broadcasted_iota(jnp.int32, (_K, 1), 0)
        priv = jnp.zeros((_K, 1), _F32)
        for k in range(_K):
            priv = priv + pri_ref[k] * jnp.where(sio8 == k, 1.0, 0.0)
        pro_ref[...] = priv * (1.0 - _MOM) + (kcnt / ic) * _MOM

        thrv = jnp.zeros((1, _K, 1), _F32)
        for k in range(_K):
            thrv = thrv + thr_ref[k] * jnp.where(kio == k, 1.0, 0.0)
        ru = jnp.clip(jnp.ceil(ind).astype(jnp.int32), 0, nm1)

        keyb = lo
        kk = keys_ref[...]
        cnt_b = _cnt(keyb)
        am = jnp.where(kk > keyb, kk, _IMAX)
        amin = jnp.min(jnp.min(am, axis=2, keepdims=True),
                       axis=0, keepdims=True)
        keyu = jnp.where(cnt_b >= (ru + 1).astype(_F32), keyb, amin)

        def _unkey(kv):
            return lax.bitcast_convert_type(
                jnp.where(kv < 0, kv ^ jnp.int32(0x7FFFFFFF), kv), _F32)

        q = 0.5 * (_unkey(keyb) + _unkey(keyu))
        tho_ref[...] = thrv * (1.0 - _MOM) + q * _MOM


def kernel(loss_field1, loss_field2, loss_presence, loss_presence_neg,
           matching, seq_lens, presence_logits,
           matching_priors, matching_thresholds):
    l1 = jnp.transpose(loss_field1, (0, 2, 3, 1))
    l2 = jnp.transpose(loss_field2, (0, 2, 3, 1))
    lp = jnp.transpose(loss_presence, (0, 2, 3, 1))
    ln = jnp.transpose(loss_presence_neg, (0, 2, 3, 1))
    mt = jnp.transpose(matching, (0, 2, 1))
    pv = jnp.transpose(presence_logits, (0, 2, 1))

    big_spec = pl.BlockSpec((1, _K, _C, _W),
                            lambda g: (g // _WPB, 0, 0, g % _WPB))
    out11 = pl.BlockSpec((1, 1), lambda g: (0, 0))
    smem = pl.BlockSpec(memory_space=pltpu.SMEM)
    f1, f2, po, pro, tho = pl.pallas_call(
        _body,
        grid=(_GRID,),
        in_specs=[
            smem, smem, smem,
            big_spec, big_spec, big_spec, big_spec,
            pl.BlockSpec((1, _K, _W), lambda g: (g // _WPB, 0, g % _WPB)),
            pl.BlockSpec((_B, _K, _L), lambda g: (0, 0, 0)),
        ],
        out_specs=[out11, out11, out11,
                   pl.BlockSpec((_K, 1), lambda g: (0, 0)),
                   pl.BlockSpec((1, _K, 1), lambda g: (0, 0, 0))],
        out_shape=[
            jax.ShapeDtypeStruct((1, 1), _F32),
            jax.ShapeDtypeStruct((1, 1), _F32),
            jax.ShapeDtypeStruct((1, 1), _F32),
            jax.ShapeDtypeStruct((_K, 1), _F32),
            jax.ShapeDtypeStruct((1, _K, 1), _F32),
        ],
        scratch_shapes=[pltpu.VMEM((5 * _K, _W), _F32),
                        pltpu.VMEM((_B, _K, _L), jnp.int32),
                        pltpu.VMEM((2, _K, 1), jnp.int32)],
    )(seq_lens, matching_priors, matching_thresholds,
      l1, l2, lp, ln, mt, pv)

    return (f1[0, 0], f2[0, 0], po[0, 0], pro[:, 0], tho[0, :, 0])
